# baseline (speedup 1.0000x reference)
import jax
import jax.numpy as jnp
from jax import lax
from jax.experimental import pallas as pl
from jax.experimental.pallas import tpu as pltpu

_VOCAB = 1000000
_EMBED = 32
_BATCH = 4096

_GB = 128
_GSTEPS = _BATCH // _GB
_BM = 512
_GPM = _BM // _GB
_MSTEPS = _BATCH // _BM
_NSTEPS = _GSTEPS + _GSTEPS + _MSTEPS


def _gstep_at(i):
  in_b = i < _GSTEPS
  r = lax.rem(i - _GSTEPS, _GPM + 1)
  j = lax.div(i - _GSTEPS, _GPM + 1)
  g_a = _GSTEPS + j * _GPM + r
  is_gather = jnp.logical_or(in_b, jnp.logical_and(i < _NSTEPS, r < _GPM))
  g = jnp.where(in_b, i, g_a)
  return g, is_gather


def _issue(sstart_ref, et_ref, bufs_ref, sems_ref, g, slot):
  base = g * _GB
  copies = []
  for k in range(_GB):
    start = pl.multiple_of(sstart_ref[base + k], 128)
    copies.append(pltpu.make_async_copy(
        et_ref.at[:, pl.ds(start, 128)],
        bufs_ref.at[slot, k],
        sems_ref.at[slot]))
  return copies


def _drain(dummy_ref, bufs_ref, sems_ref, slot):
  del dummy_ref
  for k in range(_GB):
    pltpu.make_async_copy(
        bufs_ref.at[1 - slot, k],
        bufs_ref.at[slot, k],
        sems_ref.at[slot]).wait()


def _body(sstart_ref, sshift_ref, et_ref, dummy_ref, o_ref, bufs_ref,
          a_scr, b_scr, sems_ref):
  i = pl.program_id(0)
  g_here, is_gather_here = _gstep_at(i)
  g_next, is_gather_next = _gstep_at(i + 1)

  @pl.when(i == 0)
  def _prime():
    for cp in _issue(sstart_ref, et_ref, bufs_ref, sems_ref, 0, 0):
      cp.start()

  @pl.when(jnp.logical_and(i + 1 < _NSTEPS, is_gather_next))
  def _ahead():
    for cp in _issue(sstart_ref, et_ref, bufs_ref, sems_ref,
                     g_next, lax.rem(g_next, 2)):
      cp.start()

  @pl.when(is_gather_here)
  def _gather():
    slot = lax.rem(g_here, 2)
    _drain(dummy_ref, bufs_ref, sems_ref, slot)
    kiota = lax.broadcasted_iota(jnp.int32, (_EMBED, 128), 1)
    base = g_here * _GB
    parts = []
    for k in range(_GB):
      shift = sshift_ref[base + k]
      rolled = pltpu.roll(bufs_ref[slot, k], shift, axis=1)
      parts.append(jnp.where(kiota == k, rolled, 0.0))
    while len(parts) > 1:
      parts = [parts[m] + parts[m + 1] for m in range(0, len(parts), 2)]
    acc = parts[0]
    col = pl.multiple_of(lax.rem(g_here, _GSTEPS) * _GB, 128)

    @pl.when(g_here < _GSTEPS)
    def _to_b():
      b_scr[:, pl.ds(col, _GB)] = acc

    @pl.when(g_here >= _GSTEPS)
    def _to_a():
      a_scr[pl.ds(col, _GB), :] = acc.T

  @pl.when(jnp.logical_not(is_gather_here))
  def _matmul():
    j = lax.div(i - _GSTEPS, _GPM + 1)
    row = pl.multiple_of(j * _BM, 128)
    a_tile = a_scr[pl.ds(row, _BM), :]
    o_ref[...] = lax.dot_general(
        a_tile, b_scr[...],
        dimension_numbers=(((1,), (0,)), ((), ())),
        preferred_element_type=jnp.float32)


def _out_map(i, *s):
  return (jnp.maximum(lax.div(i - _GSTEPS - _GPM, _GPM + 1), 0), 0)


_fused = pl.pallas_call(
    _body,
    grid_spec=pltpu.PrefetchScalarGridSpec(
        num_scalar_prefetch=2,
        grid=(_NSTEPS,),
        in_specs=[pl.BlockSpec(memory_space=pltpu.MemorySpace.HBM),
                  pl.BlockSpec(memory_space=pltpu.MemorySpace.HBM)],
        out_specs=pl.BlockSpec((_BM, _BATCH), _out_map),
        scratch_shapes=[
            pltpu.VMEM((2, _GB, _EMBED, 128), jnp.float32),
            pltpu.VMEM((_BATCH, _EMBED), jnp.float32),
            pltpu.VMEM((_EMBED, _BATCH), jnp.float32),
            pltpu.SemaphoreType.DMA((2,)),
        ],
    ),
    out_shape=jax.ShapeDtypeStruct((_BATCH, _BATCH), jnp.float32),
)


@jax.jit
def kernel(target, context, embeddings):
  et = embeddings.T
  idx = jnp.concatenate(
      [context.astype(jnp.int32), target.astype(jnp.int32)])
  starts = idx & ~jnp.int32(127)
  pos = jnp.arange(2 * _BATCH, dtype=jnp.int32) % _GB
  shifts = (pos - (idx & 127)) % 128
  dummy = jnp.zeros((_GB, _EMBED, 128), jnp.float32)
  return _fused(starts, shifts, et, dummy)


# speedup vs baseline: 1.0114x; 1.0114x over previous
import jax
import jax.numpy as jnp
from jax import lax
from jax.experimental import pallas as pl
from jax.experimental.pallas import tpu as pltpu

_VOCAB = 1000000
_EMBED = 32
_BATCH = 4096

_GB = 128
_GSTEPS = _BATCH // _GB
_BM = 512
_GPM = _BM // _GB
_MSTEPS = _BATCH // _BM
_NSTEPS = _GSTEPS + _GSTEPS + _MSTEPS


def _gstep_at(i):
  in_b = i < _GSTEPS
  r = lax.rem(i - _GSTEPS, _GPM + 1)
  j = lax.div(i - _GSTEPS, _GPM + 1)
  g_a = _GSTEPS + j * _GPM + r
  is_gather = jnp.logical_or(in_b, jnp.logical_and(i < _NSTEPS, r < _GPM))
  g = jnp.where(in_b, i, g_a)
  return g, is_gather


def _issue(sstart_ref, et_ref, bufs_ref, sems_ref, g, slot):
  base = g * _GB
  copies = []
  for k in range(_GB):
    start = pl.multiple_of(sstart_ref[base + k], 128)
    copies.append(pltpu.make_async_copy(
        et_ref.at[:, pl.ds(start, 128)],
        bufs_ref.at[slot, k],
        sems_ref.at[slot]))
  return copies


def _drain(dummy_ref, bufs_ref, sems_ref, slot):
  del dummy_ref
  for k in range(_GB):
    pltpu.make_async_copy(
        bufs_ref.at[1 - slot, k],
        bufs_ref.at[slot, k],
        sems_ref.at[slot]).wait()


def _body(sstart_ref, sshift_ref, et_ref, dummy_ref, o_ref, bufs_ref,
          a_scr, b_scr, sems_ref):
  i = pl.program_id(0)
  g_here, is_gather_here = _gstep_at(i)
  g_next, is_gather_next = _gstep_at(i + 1)

  @pl.when(i == 0)
  def _prime():
    for cp in _issue(sstart_ref, et_ref, bufs_ref, sems_ref, 0, 0):
      cp.start()

  @pl.when(jnp.logical_and(i + 1 < _NSTEPS, is_gather_next))
  def _ahead():
    for cp in _issue(sstart_ref, et_ref, bufs_ref, sems_ref,
                     g_next, lax.rem(g_next, 2)):
      cp.start()

  @pl.when(is_gather_here)
  def _gather():
    slot = lax.rem(g_here, 2)
    _drain(dummy_ref, bufs_ref, sems_ref, slot)
    kiota = lax.broadcasted_iota(jnp.int32, (_EMBED, 128), 1)
    base = g_here * _GB
    parts = []
    for k in range(_GB):
      shift = sshift_ref[base + k]
      rolled = pltpu.roll(bufs_ref[slot, k], shift, axis=1)
      parts.append(jnp.where(kiota == k, rolled, 0.0))
    while len(parts) > 1:
      parts = [parts[m] + parts[m + 1] for m in range(0, len(parts), 2)]
    acc = parts[0]
    col = pl.multiple_of(lax.rem(g_here, _GSTEPS) * _GB, 128)

    @pl.when(g_here < _GSTEPS)
    def _to_b():
      b_scr[:, pl.ds(col, _GB)] = acc

    @pl.when(g_here >= _GSTEPS)
    def _to_a():
      a_scr[:, pl.ds(col, _GB)] = acc

  @pl.when(jnp.logical_not(is_gather_here))
  def _matmul():
    j = lax.div(i - _GSTEPS, _GPM + 1)
    row = pl.multiple_of(j * _BM, 128)
    a_tile = a_scr[:, pl.ds(row, _BM)]
    o_ref[...] = lax.dot_general(
        a_tile, b_scr[...],
        dimension_numbers=(((0,), (0,)), ((), ())),
        preferred_element_type=jnp.float32)


def _out_map(i, *s):
  return (jnp.maximum(lax.div(i - _GSTEPS - _GPM, _GPM + 1), 0), 0)


_fused = pl.pallas_call(
    _body,
    grid_spec=pltpu.PrefetchScalarGridSpec(
        num_scalar_prefetch=2,
        grid=(_NSTEPS,),
        in_specs=[pl.BlockSpec(memory_space=pltpu.MemorySpace.HBM),
                  pl.BlockSpec(memory_space=pltpu.MemorySpace.HBM)],
        out_specs=pl.BlockSpec((_BM, _BATCH), _out_map),
        scratch_shapes=[
            pltpu.VMEM((2, _GB, _EMBED, 128), jnp.float32),
            pltpu.VMEM((_EMBED, _BATCH), jnp.float32),
            pltpu.VMEM((_EMBED, _BATCH), jnp.float32),
            pltpu.SemaphoreType.DMA((2,)),
        ],
    ),
    out_shape=jax.ShapeDtypeStruct((_BATCH, _BATCH), jnp.float32),
)


@jax.jit
def kernel(target, context, embeddings):
  et = embeddings.T
  idx = jnp.concatenate(
      [context.astype(jnp.int32), target.astype(jnp.int32)])
  starts = idx & ~jnp.int32(127)
  pos = jnp.arange(2 * _BATCH, dtype=jnp.int32) % _GB
  shifts = (pos - (idx & 127)) % 128
  dummy = jnp.zeros((_GB, _EMBED, 128), jnp.float32)
  return _fused(starts, shifts, et, dummy)


# speedup vs baseline: 1.0328x; 1.0211x over previous
import jax
import jax.numpy as jnp
from jax import lax
from jax.experimental import pallas as pl
from jax.experimental.pallas import tpu as pltpu

_VOCAB = 1000000
_EMBED = 32
_BATCH = 4096

_GB = 128
_GSTEPS = _BATCH // _GB
_BM = 512
_GPM = _BM // _GB
_MSTEPS = _BATCH // _BM
_NSTEPS = _GSTEPS + _GSTEPS + _MSTEPS


def _gstep_at(i):
  in_b = i < _GSTEPS
  r = lax.rem(i - _GSTEPS, _GPM + 1)
  j = lax.div(i - _GSTEPS, _GPM + 1)
  g_a = _GSTEPS + j * _GPM + r
  is_gather = jnp.logical_or(in_b, jnp.logical_and(i < _NSTEPS, r < _GPM))
  g = jnp.where(in_b, i, g_a)
  return g, is_gather


def _issue(sstart_ref, et_ref, bufs_ref, sems_ref, g, slot):
  base = g * _GB
  copies = []
  for k in range(_GB):
    start = pl.multiple_of(sstart_ref[base + k], 128)
    copies.append(pltpu.make_async_copy(
        et_ref.at[:, pl.ds(start, 128)],
        bufs_ref.at[slot, k],
        sems_ref.at[slot]))
  return copies


def _drain(et_ref, bufs_ref, sems_ref, slot):
  for k in range(_GB):
    pltpu.make_async_copy(
        et_ref.at[:, pl.ds(0, 128)],
        bufs_ref.at[slot, k],
        sems_ref.at[slot]).wait()


def _body(sstart_ref, sshift_ref, et_ref, o_ref, bufs_ref,
          a_scr, b_scr, sems_ref):
  i = pl.program_id(0)
  g_here, is_gather_here = _gstep_at(i)
  g_next, is_gather_next = _gstep_at(i + 1)

  @pl.when(i == 0)
  def _prime():
    for cp in _issue(sstart_ref, et_ref, bufs_ref, sems_ref, 0, 0):
      cp.start()

  @pl.when(jnp.logical_and(i + 1 < _NSTEPS, is_gather_next))
  def _ahead():
    for cp in _issue(sstart_ref, et_ref, bufs_ref, sems_ref,
                     g_next, lax.rem(g_next, 2)):
      cp.start()

  @pl.when(is_gather_here)
  def _gather():
    slot = lax.rem(g_here, 2)
    _drain(et_ref, bufs_ref, sems_ref, slot)
    kiota = lax.broadcasted_iota(jnp.int32, (_EMBED, 128), 1)
    acc = jnp.zeros((_EMBED, 128), jnp.float32)
    base = g_here * _GB
    for k in range(_GB):
      shift = sshift_ref[base + k]
      rolled = pltpu.roll(bufs_ref[slot, k], shift, axis=1)
      acc = jnp.where(kiota == k, rolled, acc)
    col = pl.multiple_of(lax.rem(g_here, _GSTEPS) * _GB, 128)

    @pl.when(g_here < _GSTEPS)
    def _to_b():
      b_scr[:, pl.ds(col, _GB)] = acc

    @pl.when(g_here >= _GSTEPS)
    def _to_a():
      a_scr[:, pl.ds(col, _GB)] = acc

  @pl.when(jnp.logical_not(is_gather_here))
  def _matmul():
    j = lax.div(i - _GSTEPS, _GPM + 1)
    row = pl.multiple_of(j * _BM, 128)
    a_tile = a_scr[:, pl.ds(row, _BM)]
    o_ref[...] = lax.dot_general(
        a_tile, b_scr[...],
        dimension_numbers=(((0,), (0,)), ((), ())),
        preferred_element_type=jnp.float32)


def _out_map(i, *s):
  return (jnp.maximum(lax.div(i - _GSTEPS - _GPM, _GPM + 1), 0), 0)


_fused = pl.pallas_call(
    _body,
    grid_spec=pltpu.PrefetchScalarGridSpec(
        num_scalar_prefetch=2,
        grid=(_NSTEPS,),
        in_specs=[pl.BlockSpec(memory_space=pltpu.MemorySpace.HBM)],
        out_specs=pl.BlockSpec((_BM, _BATCH), _out_map),
        scratch_shapes=[
            pltpu.VMEM((2, _GB, _EMBED, 128), jnp.float32),
            pltpu.VMEM((_EMBED, _BATCH), jnp.float32),
            pltpu.VMEM((_EMBED, _BATCH), jnp.float32),
            pltpu.SemaphoreType.DMA((2,)),
        ],
    ),
    out_shape=jax.ShapeDtypeStruct((_BATCH, _BATCH), jnp.float32),
)


@jax.jit
def kernel(target, context, embeddings):
  et = embeddings.T
  idx = jnp.concatenate(
      [context.astype(jnp.int32), target.astype(jnp.int32)])
  starts = idx & ~jnp.int32(127)
  pos = jnp.arange(2 * _BATCH, dtype=jnp.int32) % _GB
  shifts = (pos - (idx & 127)) % 128
  return _fused(starts, shifts, et)
